# Initial kernel scaffold; baseline (speedup 1.0000x reference)
import jax
import jax.numpy as jnp
from jax.experimental import pallas as pl


def kernel(x, pos_table):
    raise NotImplementedError("write your pallas kernel here")


# speedup vs baseline: 1.4587x; 1.4587x over previous
import jax
import jax.numpy as jnp
from jax.experimental import pallas as pl

S_BLK = 256


def _add_body(x_ref, t_ref, o_ref):
    o_ref[...] = x_ref[...] + t_ref[...][None, :, :]


def kernel(x, pos_table):
    batch, seq_len, d_model = x.shape
    table = pos_table[:seq_len]
    grid = (seq_len // S_BLK, batch)
    return pl.pallas_call(
        _add_body,
        grid=grid,
        in_specs=[
            pl.BlockSpec((1, S_BLK, d_model), lambda i, j: (j, i, 0)),
            pl.BlockSpec((S_BLK, d_model), lambda i, j: (i, 0)),
        ],
        out_specs=pl.BlockSpec((1, S_BLK, d_model), lambda i, j: (j, i, 0)),
        out_shape=jax.ShapeDtypeStruct((batch, seq_len, d_model), x.dtype),
    )(x, table)


# speedup vs baseline: 1.9248x; 1.3195x over previous
import jax
import jax.numpy as jnp
from jax.experimental import pallas as pl

S_BLK = 512


def _add_body(x_ref, t_ref, o_ref):
    o_ref[...] = x_ref[...] + t_ref[...][None, :, :]


def kernel(x, pos_table):
    batch, seq_len, d_model = x.shape
    table = pos_table[:seq_len]
    grid = (seq_len // S_BLK, batch)
    return pl.pallas_call(
        _add_body,
        grid=grid,
        in_specs=[
            pl.BlockSpec((1, S_BLK, d_model), lambda i, j: (j, i, 0)),
            pl.BlockSpec((S_BLK, d_model), lambda i, j: (i, 0)),
        ],
        out_specs=pl.BlockSpec((1, S_BLK, d_model), lambda i, j: (j, i, 0)),
        out_shape=jax.ShapeDtypeStruct((batch, seq_len, d_model), x.dtype),
    )(x, table)


# speedup vs baseline: 2.1105x; 1.0965x over previous
import jax
import jax.numpy as jnp
from jax.experimental import pallas as pl

S_BLK = 1024


def _add_body(x_ref, t_ref, o_ref):
    o_ref[...] = x_ref[...] + t_ref[...][None, :, :]


def kernel(x, pos_table):
    batch, seq_len, d_model = x.shape
    table = pos_table[:seq_len]
    grid = (seq_len // S_BLK, batch)
    return pl.pallas_call(
        _add_body,
        grid=grid,
        in_specs=[
            pl.BlockSpec((1, S_BLK, d_model), lambda i, j: (j, i, 0)),
            pl.BlockSpec((S_BLK, d_model), lambda i, j: (i, 0)),
        ],
        out_specs=pl.BlockSpec((1, S_BLK, d_model), lambda i, j: (j, i, 0)),
        out_shape=jax.ShapeDtypeStruct((batch, seq_len, d_model), x.dtype),
    )(x, table)


# speedup vs baseline: 2.2324x; 1.0578x over previous
import jax
import jax.numpy as jnp
from jax.experimental import pallas as pl

S_BLK = 2048


def _add_body(x_ref, t_ref, o_ref):
    o_ref[...] = x_ref[...] + t_ref[...][None, :, :]


def kernel(x, pos_table):
    batch, seq_len, d_model = x.shape
    table = pos_table[:seq_len]
    grid = (seq_len // S_BLK, batch)
    return pl.pallas_call(
        _add_body,
        grid=grid,
        in_specs=[
            pl.BlockSpec((1, S_BLK, d_model), lambda i, j: (j, i, 0)),
            pl.BlockSpec((S_BLK, d_model), lambda i, j: (i, 0)),
        ],
        out_specs=pl.BlockSpec((1, S_BLK, d_model), lambda i, j: (j, i, 0)),
        out_shape=jax.ShapeDtypeStruct((batch, seq_len, d_model), x.dtype),
    )(x, table)
